# Initial kernel scaffold; baseline (speedup 1.0000x reference)
import jax
import jax.numpy as jnp
from jax.experimental import pallas as pl


def kernel(y_1, y_2, y_noise, forget_rate, ind, noise_or_not):
    raise NotImplementedError("write your pallas kernel here")


# speedup vs baseline: 2.5872x; 2.5872x over previous
import functools

import jax
import jax.numpy as jnp
from jax import lax
from jax.experimental import pallas as pl
from jax.experimental.pallas import tpu as pltpu
from jax.experimental.pallas import tpu_sc as plsc

B = 16384
C = 1000
BR = 256
NB = B // BR

_NC = 2
_NS = 16
_NW = _NC * _NS
_BPW = B // _NW


def _ce_body(lab_ref, y1_ref, y2_ref, o1_ref, o2_ref):
    lab = lab_ref[0]
    col = lax.broadcasted_iota(jnp.int32, (BR, C), 1)
    onehot = col == lab
    x1 = y1_ref[...]
    s1 = jnp.sum(jnp.exp(x1), axis=1, keepdims=True)
    g1 = jnp.sum(jnp.where(onehot, x1, 0.0), axis=1, keepdims=True)
    o1_ref[0] = jnp.log(s1) - g1
    x2 = y2_ref[...]
    s2 = jnp.sum(jnp.exp(x2), axis=1, keepdims=True)
    g2 = jnp.sum(jnp.where(onehot, x2, 0.0), axis=1, keepdims=True)
    o2_ref[0] = jnp.log(s2) - g2


def _ce_losses(y_1, y_2, y_noise, interpret=False):
    labs = y_noise.reshape(NB, BR, 1)
    out_shape = jax.ShapeDtypeStruct((NB, BR, 1), jnp.float32)
    l1, l2 = pl.pallas_call(
        _ce_body,
        grid=(NB,),
        in_specs=[
            pl.BlockSpec((1, BR, 1), lambda i: (i, 0, 0)),
            pl.BlockSpec((BR, C), lambda i: (i, 0)),
            pl.BlockSpec((BR, C), lambda i: (i, 0)),
        ],
        out_specs=[
            pl.BlockSpec((1, BR, 1), lambda i: (i, 0, 0)),
            pl.BlockSpec((1, BR, 1), lambda i: (i, 0, 0)),
        ],
        out_shape=[out_shape, out_shape],
        compiler_params=pltpu.CompilerParams(
            dimension_semantics=("arbitrary",),
        ),
        interpret=interpret,
    )(labs, y_1, y_2)
    return l1.reshape(B), l2.reshape(B)


def _sc_gather(noise_or_not, ind):
    mesh = plsc.VectorSubcoreMesh(core_axis_name="c", subcore_axis_name="s")

    @functools.partial(
        pl.kernel,
        out_type=jax.ShapeDtypeStruct((B,), jnp.int32),
        mesh=mesh,
        scratch_types=[
            pltpu.VMEM((_BPW,), jnp.int32),
            pltpu.VMEM((_BPW,), jnp.int32),
            pltpu.SemaphoreType.DMA,
        ],
    )
    def gather_kernel(noise_hbm, ind_hbm, out_hbm, idx_v, vals_v, sem):
        wid = lax.axis_index("s") * _NC + lax.axis_index("c")
        base = wid * _BPW
        pltpu.sync_copy(ind_hbm.at[pl.ds(base, _BPW)], idx_v)
        pltpu.async_copy(noise_hbm.at[idx_v], vals_v, sem).wait()
        pltpu.sync_copy(vals_v, out_hbm.at[pl.ds(base, _BPW)])

    return gather_kernel(noise_or_not, ind)


def _sel_body(k_ref, l1_ref, l2_ref, nn_ref, o1_ref, o2_ref, o3_ref, o4_ref):
    _MSB = jnp.int32(-2147483648)
    k = k_ref[0]
    l1 = l1_ref[...]
    l2 = l2_ref[...]
    nn = nn_ref[...]
    bits1 = lax.bitcast_convert_type(l1, jnp.int32)
    bits2 = lax.bitcast_convert_type(l2, jnp.int32)
    key1 = bits1 ^ ((bits1 >> 31) & jnp.int32(0x7FFFFFFF))
    key2 = bits2 ^ ((bits2 >> 31) & jnp.int32(0x7FFFFFFF))

    def vstep(i, carry):
        t1, t2 = carry
        bit = jnp.left_shift(jnp.int32(1), jnp.int32(31) - i)
        c1 = t1 | bit
        c2 = t2 | bit
        cnt1 = jnp.sum((key1 < (c1 ^ _MSB)).astype(jnp.int32))
        cnt2 = jnp.sum((key2 < (c2 ^ _MSB)).astype(jnp.int32))
        return (jnp.where(cnt1 < k, c1, t1), jnp.where(cnt2 < k, c2, t2))

    t1b, t2b = lax.fori_loop(0, 32, vstep, (jnp.int32(0), jnp.int32(0)))
    t1 = t1b ^ _MSB
    t2 = t2b ^ _MSB

    less1 = key1 < t1
    less2 = key2 < t2
    eq1 = key1 == t1
    eq2 = key2 == t2
    need1 = k - jnp.sum(less1.astype(jnp.int32))
    need2 = k - jnp.sum(less2.astype(jnp.int32))

    rows = l1.shape[0]
    cols = l1.shape[1]
    idx = (lax.broadcasted_iota(jnp.int32, (rows, cols), 0) * cols
           + lax.broadcasted_iota(jnp.int32, (rows, cols), 1))

    def istep(i, carry):
        j1, j2 = carry
        bit = jnp.left_shift(jnp.int32(1), jnp.int32(13) - i)
        c1 = j1 | bit
        c2 = j2 | bit
        cnt1 = jnp.sum((eq1 & (idx < c1)).astype(jnp.int32))
        cnt2 = jnp.sum((eq2 & (idx < c2)).astype(jnp.int32))
        return (jnp.where(cnt1 < need1, c1, j1), jnp.where(cnt2 < need2, c2, j2))

    j1, j2 = lax.fori_loop(0, 14, istep, (jnp.int32(0), jnp.int32(0)))

    kept1 = less1 | (eq1 & (idx <= j1))
    kept2 = less2 | (eq2 & (idx <= j2))
    kf = k.astype(jnp.float32)
    o1_ref[0, 0] = jnp.sum(jnp.where(kept2, l1, 0.0)) / kf
    o2_ref[0, 0] = jnp.sum(jnp.where(kept1, l2, 0.0)) / kf
    o3_ref[0, 0] = jnp.sum(jnp.where(kept1, nn, 0.0)) / kf
    o4_ref[0, 0] = jnp.sum(jnp.where(kept2, nn, 0.0)) / kf


def _select(loss_1, loss_2, nn_f, k_arr, interpret=False):
    scal = jax.ShapeDtypeStruct((1, 1), jnp.float32)
    return pl.pallas_call(
        _sel_body,
        in_specs=[
            pl.BlockSpec(memory_space=pltpu.SMEM),
            pl.BlockSpec((128, 128), lambda: (0, 0)),
            pl.BlockSpec((128, 128), lambda: (0, 0)),
            pl.BlockSpec((128, 128), lambda: (0, 0)),
        ],
        out_specs=[
            pl.BlockSpec(memory_space=pltpu.SMEM),
            pl.BlockSpec(memory_space=pltpu.SMEM),
            pl.BlockSpec(memory_space=pltpu.SMEM),
            pl.BlockSpec(memory_space=pltpu.SMEM),
        ],
        out_shape=[scal, scal, scal, scal],
        interpret=interpret,
    )(k_arr, loss_1.reshape(128, 128), loss_2.reshape(128, 128), nn_f)


def kernel(y_1, y_2, y_noise, forget_rate, ind, noise_or_not):
    loss_1, loss_2 = _ce_losses(y_1, y_2, y_noise)
    nn = _sc_gather(noise_or_not, ind)
    nn_f = nn.astype(jnp.float32).reshape(128, 128)
    remember_rate = 1.0 - forget_rate
    num_remember = jnp.floor(remember_rate * B).astype(jnp.int32)
    k_arr = num_remember.reshape(1)
    o1, o2, o3, o4 = _select(loss_1, loss_2, nn_f, k_arr)
    return (o1[0, 0], o2[0, 0], o3[0, 0], o4[0, 0])
